# baseline (device time: 227746 ns/iter reference)
import functools

import jax
import jax.numpy as jnp
from jax import lax
from jax.experimental import pallas as pl
from jax.experimental.pallas import tpu as pltpu

N_DEV = 16
B = 2
S = 256
SKV = N_DEV * S
H = 4
DH = 64
HD = H * DH
DKV = 2 * HD


def kernel(x, Wq, K_ext, V_ext, Wo):
    kv = jnp.concatenate(
        [K_ext.reshape(B, S, HD), V_ext.reshape(B, S, HD)], axis=-1
    )

    def body(x_ref, wq_ref, kv_ref, wo_ref, out_ref, kvbuf, send_sems, recv_sems):
        my = lax.axis_index("i")
        left = lax.rem(my - 1 + N_DEV, N_DEV)
        right = lax.rem(my + 1, N_DEV)

        barrier_sem = pltpu.get_barrier_semaphore()
        for nbr in (left, right):
            pl.semaphore_signal(
                barrier_sem, inc=1,
                device_id=(nbr,), device_id_type=pl.DeviceIdType.MESH,
            )
        pl.semaphore_wait(barrier_sem, 2)

        kvbuf[:, pl.ds(my * S, S), :] = kv_ref[...]

        for h in range(N_DEV - 1):
            origin = lax.rem(my - h + N_DEV, N_DEV)
            chunk = kvbuf.at[:, pl.ds(origin * S, S), :]
            rdma = pltpu.make_async_remote_copy(
                src_ref=chunk,
                dst_ref=chunk,
                send_sem=send_sems.at[h],
                recv_sem=recv_sems.at[h],
                device_id=(right,),
                device_id_type=pl.DeviceIdType.MESH,
            )
            rdma.start()
            rdma.wait()

        for b in range(B):
            qb = jnp.dot(
                x_ref[b], wq_ref[...], preferred_element_type=jnp.float32
            )
            ctx_cols = []
            for hh in range(H):
                q = qb[:, hh * DH:(hh + 1) * DH]
                k = kvbuf[b, :, hh * DH:(hh + 1) * DH]
                v = kvbuf[b, :, HD + hh * DH:HD + (hh + 1) * DH]
                s = lax.dot_general(
                    q, k, (((1,), (1,)), ((), ())),
                    preferred_element_type=jnp.float32,
                ) * 0.125
                qi = my * S + lax.broadcasted_iota(jnp.int32, (S, SKV), 0)
                ki = lax.broadcasted_iota(jnp.int32, (S, SKV), 1)
                mask = (jnp.abs(qi - ki) <= 128) | (ki < 32) | (qi < 32)
                s = jnp.where(mask, s, -1e9)
                m = jnp.max(s, axis=-1, keepdims=True)
                w = jnp.exp(s - m)
                w = w / jnp.sum(w, axis=-1, keepdims=True)
                ctx_cols.append(
                    jnp.dot(w, v, preferred_element_type=jnp.float32)
                )
            ctxb = jnp.concatenate(ctx_cols, axis=-1)
            out_ref[b] = jnp.dot(
                ctxb, wo_ref[...], preferred_element_type=jnp.float32
            )

        @functools.partial(
            pl.run_scoped, second_barrier=pltpu.SemaphoreType.REGULAR
        )
        def _(second_barrier):
            for nbr in (left, right):
                pl.semaphore_signal(
                    second_barrier, inc=1,
                    device_id=(nbr,), device_id_type=pl.DeviceIdType.MESH,
                )
            pl.semaphore_wait(second_barrier, 2)

    return pl.pallas_call(
        body,
        out_shape=jax.ShapeDtypeStruct((B, S, 2 * HD), jnp.float32),
        in_specs=[
            pl.BlockSpec(memory_space=pltpu.VMEM),
            pl.BlockSpec(memory_space=pltpu.VMEM),
            pl.BlockSpec(memory_space=pltpu.VMEM),
            pl.BlockSpec(memory_space=pltpu.VMEM),
        ],
        out_specs=pl.BlockSpec(memory_space=pltpu.VMEM),
        scratch_shapes=[
            pltpu.VMEM((B, SKV, DKV), jnp.float32),
            pltpu.SemaphoreType.DMA((N_DEV - 1,)),
            pltpu.SemaphoreType.DMA((N_DEV - 1,)),
        ],
        compiler_params=pltpu.CompilerParams(collective_id=0),
    )(x, Wq, kv, Wo)


# device time: 54398 ns/iter; 4.1867x vs baseline; 4.1867x over previous
import jax
import jax.numpy as jnp
from jax import lax
from jax.experimental import pallas as pl
from jax.experimental.pallas import tpu as pltpu

N_DEV = 16
B = 2
S = 256
H = 4
DH = 64
HD = H * DH
DKV = 2 * HD
HALO = 128
G = 32
KTOT = G + HALO + S + HALO
RED_C = HD + 2 * H
F32 = jnp.float32


def kernel(x, Wq, K_ext, V_ext, Wo):
    kv = jnp.concatenate(
        [K_ext.reshape(B, S, HD), V_ext.reshape(B, S, HD)], axis=-1
    )

    def body(x_ref, wq_ref, kv_ref, wo_ref, out_ref,
             lh_buf, rh_buf, bc_buf, red_send, red_recv,
             hs_sems, hr_sems, bc_ssems, bc_rsem, rd_ssem, rd_rsems):
        my = lax.axis_index("i")
        left = lax.rem(my - 1 + N_DEV, N_DEV)
        right = lax.rem(my + 1, N_DEV)

        rdma_a = pltpu.make_async_remote_copy(
            src_ref=kv_ref.at[:, pl.ds(S - HALO, HALO), :],
            dst_ref=lh_buf,
            send_sem=hs_sems.at[0], recv_sem=hr_sems.at[0],
            device_id=(right,), device_id_type=pl.DeviceIdType.MESH,
        )
        rdma_a.start()
        rdma_b = pltpu.make_async_remote_copy(
            src_ref=kv_ref.at[:, pl.ds(0, HALO), :],
            dst_ref=rh_buf,
            send_sem=hs_sems.at[1], recv_sem=hr_sems.at[1],
            device_id=(left,), device_id_type=pl.DeviceIdType.MESH,
        )
        rdma_b.start()

        q = [
            jnp.dot(x_ref[b], wq_ref[...], preferred_element_type=F32)
            for b in range(B)
        ]

        def bc_rdma(t):
            return pltpu.make_async_remote_copy(
                src_ref=bc_buf, dst_ref=bc_buf,
                send_sem=bc_ssems.at[max(t - 1, 0)], recv_sem=bc_rsem,
                device_id=(t,), device_id_type=pl.DeviceIdType.MESH,
            )

        @pl.when(my == 0)
        def _():
            bc_buf[:, :, 0:DKV] = kv_ref[:, 0:G, :]
            for b in range(B):
                bc_buf[b, :, DKV:DKV + HD] = q[b][0:G, :]
            for t in range(1, N_DEV):
                bc_rdma(t).start()

        @pl.when(my != 0)
        def _():
            bc_rdma(0).wait_recv()

        own_m = [[None] * H for _ in range(B)]
        own_l = [[None] * H for _ in range(B)]
        own_o = [[None] * H for _ in range(B)]
        for b in range(B):
            for h in range(H):
                q0 = bc_buf[b, :, DKV + h * DH:DKV + (h + 1) * DH]
                ko = kv_ref[b, :, h * DH:(h + 1) * DH]
                vo = kv_ref[b, :, HD + h * DH:HD + (h + 1) * DH]
                s = lax.dot_general(
                    q0, ko, (((1,), (1,)), ((), ())),
                    preferred_element_type=F32,
                ) * 0.125
                mm = jnp.max(s, axis=-1, keepdims=True)
                w = jnp.exp(s - mm)
                own_m[b][h] = mm
                own_l[b][h] = jnp.sum(w, axis=-1, keepdims=True)
                own_o[b][h] = jnp.dot(
                    w, vo, preferred_element_type=F32
                )

        def rd_rdma(slot):
            return pltpu.make_async_remote_copy(
                src_ref=red_send, dst_ref=red_recv.at[slot],
                send_sem=rd_ssem, recv_sem=rd_rsems.at[slot],
                device_id=(0,), device_id_type=pl.DeviceIdType.MESH,
            )

        @pl.when(my != 0)
        def _():
            for b in range(B):
                for h in range(H):
                    red_send[b, :, h * DH:(h + 1) * DH] = own_o[b][h]
                    red_send[b, :, HD + h:HD + h + 1] = own_m[b][h]
                    red_send[b, :, HD + H + h:HD + H + h + 1] = own_l[b][h]
            rd_rdma(my - 1).start()

        rdma_a.wait_recv()
        rdma_b.wait_recv()

        riota = lax.broadcasted_iota(jnp.int32, (S, KTOT), 0)
        ciota = lax.broadcasted_iota(jnp.int32, (S, KTOT), 1)
        jl = ciota - G
        jo = ciota - (G + HALO)
        jr = ciota - (G + HALO + S)
        m_g = (ciota < G) & (my >= 1)
        m_l = (ciota >= G) & (ciota < G + HALO) & (my >= 1) & (jl >= riota)
        m_o = (
            (ciota >= G + HALO) & (ciota < G + HALO + S)
            & ((jnp.abs(riota - jo) <= HALO) | ((my == 0) & (jo < G)))
        )
        m_r = (
            (ciota >= G + HALO + S) & (my <= N_DEV - 2)
            & (jr <= riota - HALO)
        )
        mask = (m_g | m_l | m_o | m_r) & ((my >= 1) | (riota >= G))

        ctx_cols = [[None] * H for _ in range(B)]
        for b in range(B):
            for h in range(H):
                kc, vc = h * DH, HD + h * DH
                k_all = jnp.concatenate([
                    bc_buf[b, :, kc:kc + DH],
                    lh_buf[b, :, kc:kc + DH],
                    kv_ref[b, :, kc:kc + DH],
                    rh_buf[b, :, kc:kc + DH],
                ], axis=0)
                v_all = jnp.concatenate([
                    bc_buf[b, :, vc:vc + DH],
                    lh_buf[b, :, vc:vc + DH],
                    kv_ref[b, :, vc:vc + DH],
                    rh_buf[b, :, vc:vc + DH],
                ], axis=0)
                s = lax.dot_general(
                    q[b][:, kc:kc + DH], k_all, (((1,), (1,)), ((), ())),
                    preferred_element_type=F32,
                ) * 0.125
                s = jnp.where(mask, s, -1e9)
                mm = jnp.max(s, axis=-1, keepdims=True)
                w = jnp.exp(s - mm)
                w = w / jnp.sum(w, axis=-1, keepdims=True)
                ctx_cols[b][h] = jnp.dot(
                    w, v_all, preferred_element_type=F32
                )

        @pl.when(my == 0)
        def _():
            for t in range(1, N_DEV):
                rd_rdma(t - 1).wait_recv()

        arr = red_recv[...]
        rows = lax.broadcasted_iota(jnp.int32, (S, DH), 0)
        for b in range(B):
            for h in range(H):
                st_m = jnp.concatenate(
                    [own_m[b][h][None], arr[:, b, :, HD + h:HD + h + 1]],
                    axis=0,
                )
                st_l = jnp.concatenate(
                    [own_l[b][h][None],
                     arr[:, b, :, HD + H + h:HD + H + h + 1]],
                    axis=0,
                )
                st_o = jnp.concatenate(
                    [own_o[b][h][None], arr[:, b, :, h * DH:(h + 1) * DH]],
                    axis=0,
                )
                mx = jnp.max(st_m, axis=0)
                sc = jnp.exp(st_m - mx[None])
                lsum = jnp.sum(sc * st_l, axis=0)
                osum = jnp.sum(sc * st_o, axis=0)
                gctx = osum / lsum
                gfull = jnp.concatenate(
                    [gctx, jnp.zeros((S - G, DH), F32)], axis=0
                )
                ctx_cols[b][h] = jnp.where(
                    (my == 0) & (rows < G), gfull, ctx_cols[b][h]
                )

        for b in range(B):
            ctxb = jnp.concatenate(ctx_cols[b], axis=-1)
            out_ref[b] = jnp.dot(
                ctxb, wo_ref[...], preferred_element_type=F32
            )

        rdma_a.wait_send()
        rdma_b.wait_send()

        @pl.when(my == 0)
        def _():
            for t in range(1, N_DEV):
                bc_rdma(t).wait_send()

        @pl.when(my != 0)
        def _():
            rd_rdma(my - 1).wait_send()

    return pl.pallas_call(
        body,
        out_shape=jax.ShapeDtypeStruct((B, S, 2 * HD), jnp.float32),
        in_specs=[
            pl.BlockSpec(memory_space=pltpu.VMEM),
            pl.BlockSpec(memory_space=pltpu.VMEM),
            pl.BlockSpec(memory_space=pltpu.VMEM),
            pl.BlockSpec(memory_space=pltpu.VMEM),
        ],
        out_specs=pl.BlockSpec(memory_space=pltpu.VMEM),
        scratch_shapes=[
            pltpu.VMEM((B, HALO, DKV), jnp.float32),
            pltpu.VMEM((B, HALO, DKV), jnp.float32),
            pltpu.VMEM((B, G, DKV + HD), jnp.float32),
            pltpu.VMEM((B, G, RED_C), jnp.float32),
            pltpu.VMEM((N_DEV - 1, B, G, RED_C), jnp.float32),
            pltpu.SemaphoreType.DMA((2,)),
            pltpu.SemaphoreType.DMA((2,)),
            pltpu.SemaphoreType.DMA((N_DEV - 1,)),
            pltpu.SemaphoreType.DMA,
            pltpu.SemaphoreType.DMA,
            pltpu.SemaphoreType.DMA((N_DEV - 1,)),
        ],
    )(x, Wq, kv, Wo)
